# baseline (speedup 1.0000x reference)
import functools

import jax
import jax.numpy as jnp
from jax import lax
from jax.experimental import pallas as pl
from jax.experimental.pallas import tpu as pltpu
from jax.experimental.pallas import tpu_sc as plsc

D_MODEL = 64
SCALE = 8.0
LANES = 16
IB = 128
NBUF = 2


@functools.lru_cache(maxsize=None)
def _build(S, N):
    info = plsc.get_sparse_core_info()
    NC, NS = info.num_cores, info.num_subcores
    NW = NC * NS
    nblk = N // IB
    tasks = S * nblk
    t_per_w = tasks // NW
    assert t_per_w % NBUF == 0
    mesh = plsc.VectorSubcoreMesh(core_axis_name="c", subcore_axis_name="s")

    @functools.partial(
        pl.kernel,
        mesh=mesh,
        out_type=jax.ShapeDtypeStruct((S * D_MODEL, N), jnp.float32),
        scratch_types=[
            pltpu.VMEM((IB,), jnp.int32),
            pltpu.VMEM((IB,), jnp.int32),
            pltpu.VMEM((IB, D_MODEL), jnp.float32),
            pltpu.VMEM((IB, D_MODEL), jnp.float32),
            pltpu.VMEM((D_MODEL, IB), jnp.float32),
            pltpu.VMEM((D_MODEL, IB), jnp.float32),
            pltpu.SemaphoreType.DMA,
            pltpu.SemaphoreType.DMA,
            pltpu.SemaphoreType.DMA,
        ],
        compiler_params=pltpu.CompilerParams(use_tc_tiling_on_sc=False,
                                             needs_layout_passes=False),
    )
    def k(xt_hbm, table_hbm, out_hbm, idx0, idx1, rows0, rows1, rt0, rt1,
          isem, gsem, wsem):
        idx_v = (idx0, idx1)
        rows_v = (rows0, rows1)
        rt_v = (rt0, rt1)
        wid = lax.axis_index("s") * NC + lax.axis_index("c")
        t0 = wid * t_per_w

        def task_jc(g):
            t = t0 + g
            return t // nblk, (t % nblk) * IB

        def fire_idx(g, b):
            j, i0 = task_jc(g)
            pltpu.async_copy(xt_hbm.at[j, pl.ds(i0, IB)], idx_v[b], isem)

        def fire_gather(b):
            pltpu.async_copy(table_hbm.at[idx_v[b]], rows_v[b], gsem)

        def drain_rows(b):
            pltpu.make_async_copy(table_hbm.at[pl.ds(0, IB)], rows_v[b],
                                  gsem).wait()

        def drain_wr(b):
            pltpu.make_async_copy(
                out_hbm.at[pl.ds(0, D_MODEL), pl.ds(0, IB)], rt_v[b],
                wsem).wait()

        def drain_idx(b):
            pltpu.make_async_copy(xt_hbm.at[0, pl.ds(0, IB)], idx_v[b],
                                  isem).wait()

        fire_idx(0, 0)
        fire_idx(1, 1)
        drain_idx(0)
        fire_gather(0)

        def outer(g2, carry):
            for b in range(NBUF):
                g = g2 * NBUF + b
                nb = (b + 1) % NBUF
                drain_rows(b)

                @pl.when(g + 2 < t_per_w)
                def _():
                    fire_idx(g + 2, b)

                @pl.when(g + 1 < t_per_w)
                def _():
                    drain_idx(nb)
                    fire_gather(nb)

                @pl.when(g >= NBUF)
                def _():
                    drain_wr(b)

                rows_b = rows_v[b]
                rt_b = rt_v[b]

                def tr_body(c, c2):
                    cvec = jnp.full((LANES,), c, jnp.int32)
                    for grp in range(IB // LANES):
                        ivec = lax.iota(jnp.int32, LANES) + (grp * LANES)
                        v = plsc.load_gather(rows_b, [ivec, cvec])
                        rt_b[c, pl.ds(grp * LANES, LANES)] = v * SCALE
                    return c2

                lax.fori_loop(0, D_MODEL, tr_body, 0)

                j, i0 = task_jc(g)
                pltpu.async_copy(
                    rt_b,
                    out_hbm.at[pl.ds(j * D_MODEL, D_MODEL), pl.ds(i0, IB)],
                    wsem,
                )
            return carry

        lax.fori_loop(0, t_per_w // NBUF, outer, 0)
        drain_wr(0)
        drain_wr(1)

    return k


def kernel(x, table):
    N, S = x.shape
    xt = jnp.transpose(x).astype(jnp.int32)
    out2 = _build(S, N)(xt, table)
    out3 = out2.reshape(S, D_MODEL, N)
    return jnp.transpose(out3, (2, 0, 1))


# speedup vs baseline: 1.3078x; 1.3078x over previous
import functools

import jax
import jax.numpy as jnp
from jax import lax
from jax.experimental import pallas as pl
from jax.experimental.pallas import tpu as pltpu
from jax.experimental.pallas import tpu_sc as plsc

D_MODEL = 64
DPAD = 128
SCALE = 8.0
LANES = 16
IB = 128
NBUF = 2


@functools.lru_cache(maxsize=None)
def _build(S, N):
    info = plsc.get_sparse_core_info()
    NC, NS = info.num_cores, info.num_subcores
    NW = NC * NS
    nblk = N // IB
    t_per_w = S * nblk // NW
    assert t_per_w % NBUF == 0
    mesh = plsc.VectorSubcoreMesh(core_axis_name="c", subcore_axis_name="s")

    @functools.partial(
        pl.kernel,
        mesh=mesh,
        out_type=jax.ShapeDtypeStruct((S * D_MODEL, N), jnp.float32),
        scratch_types=[
            pltpu.VMEM((IB,), jnp.int32),
            pltpu.VMEM((IB,), jnp.int32),
            pltpu.VMEM((IB, DPAD), jnp.float32),
            pltpu.VMEM((IB, DPAD), jnp.float32),
            pltpu.VMEM((D_MODEL, IB), jnp.float32),
            pltpu.VMEM((D_MODEL, IB), jnp.float32),
            pltpu.SemaphoreType.DMA,
            pltpu.SemaphoreType.DMA,
            pltpu.SemaphoreType.DMA,
        ],
        compiler_params=pltpu.CompilerParams(needs_layout_passes=False),
    )
    def k(xt_hbm, tp_hbm, out_hbm, idx0, idx1, rows0, rows1, rt0, rt1,
          isem, gsem, wsem):
        idx_v = (idx0, idx1)
        rows_v = (rows0, rows1)
        rt_v = (rt0, rt1)
        wid = lax.axis_index("s") * NC + lax.axis_index("c")
        t0 = wid * t_per_w

        def task_jc(g):
            t = t0 + g
            return t // nblk, (t % nblk) * IB

        def fire_idx(g, b):
            j, i0 = task_jc(g)
            pltpu.async_copy(xt_hbm.at[j, pl.ds(i0, IB)], idx_v[b], isem)

        def fire_gather(b):
            pltpu.async_copy(tp_hbm.at[idx_v[b]], rows_v[b], gsem)

        def drain_rows(b):
            pltpu.make_async_copy(tp_hbm.at[pl.ds(0, IB)], rows_v[b],
                                  gsem).wait()

        def drain_wr(b):
            pltpu.make_async_copy(
                out_hbm.at[pl.ds(0, D_MODEL), pl.ds(0, IB)], rt_v[b],
                wsem).wait()

        def drain_idx(b):
            pltpu.make_async_copy(xt_hbm.at[0, pl.ds(0, IB)], idx_v[b],
                                  isem).wait()

        fire_idx(0, 0)
        fire_idx(1, 1)
        drain_idx(0)
        fire_gather(0)

        def outer(g2, carry):
            for b in range(NBUF):
                g = g2 * NBUF + b
                nb = (b + 1) % NBUF
                drain_rows(b)

                @pl.when(g + 2 < t_per_w)
                def _():
                    fire_idx(g + 2, b)

                @pl.when(g + 1 < t_per_w)
                def _():
                    drain_idx(nb)
                    fire_gather(nb)

                @pl.when(g >= NBUF)
                def _():
                    drain_wr(b)

                rows_b = rows_v[b]
                rt_b = rt_v[b]
                cvecs = [lax.iota(jnp.int32, LANES) + (cb * LANES)
                         for cb in range(D_MODEL // LANES)]

                def tr_body(i, c2):
                    ivec = jnp.full((LANES,), i, jnp.int32)
                    for cb in range(D_MODEL // LANES):
                        v = rows_b[i, pl.ds(cb * LANES, LANES)]
                        plsc.store_scatter(rt_b, [cvecs[cb], ivec],
                                           v * SCALE)
                    return c2

                lax.fori_loop(0, IB, tr_body, 0, unroll=8)

                j, i0 = task_jc(g)
                pltpu.async_copy(
                    rt_b,
                    out_hbm.at[pl.ds(j * D_MODEL, D_MODEL), pl.ds(i0, IB)],
                    wsem,
                )
            return carry

        lax.fori_loop(0, t_per_w // NBUF, outer, 0)
        drain_wr(0)
        drain_wr(1)

    return k


def kernel(x, table):
    N, S = x.shape
    xt = jnp.transpose(x).astype(jnp.int32)
    tp = jnp.pad(table, ((0, 0), (0, DPAD - D_MODEL)))
    out2 = _build(S, N)(xt, tp)
    out3 = out2.reshape(S, D_MODEL, N)
    return jnp.transpose(out3, (2, 0, 1))


# speedup vs baseline: 1.7376x; 1.3286x over previous
import functools

import jax
import jax.numpy as jnp
from jax import lax
from jax.experimental import pallas as pl
from jax.experimental.pallas import tpu as pltpu
from jax.experimental.pallas import tpu_sc as plsc

D_MODEL = 64
DPAD = 128
SCALE = 8.0
LANES = 16
IB = 128
NBUF = 2


@functools.lru_cache(maxsize=None)
def _build(S, N):
    info = plsc.get_sparse_core_info()
    NC, NS = info.num_cores, info.num_subcores
    NW = NC * NS
    nblk = N // IB
    t_per_w = S * nblk // NW
    assert t_per_w % NBUF == 0
    mesh = plsc.VectorSubcoreMesh(core_axis_name="c", subcore_axis_name="s")

    @functools.partial(
        pl.kernel,
        mesh=mesh,
        out_type=jax.ShapeDtypeStruct((S * D_MODEL, N), jnp.float32),
        scratch_types=[
            pltpu.VMEM((IB,), jnp.int32),
            pltpu.VMEM((IB,), jnp.int32),
            pltpu.VMEM((IB, DPAD), jnp.float32),
            pltpu.VMEM((IB, DPAD), jnp.float32),
            pltpu.VMEM((D_MODEL, IB), jnp.float32),
            pltpu.VMEM((D_MODEL, IB), jnp.float32),
            pltpu.SemaphoreType.DMA,
            pltpu.SemaphoreType.DMA,
            pltpu.SemaphoreType.DMA,
        ],
        compiler_params=pltpu.CompilerParams(needs_layout_passes=False),
    )
    def k(xt_hbm, tp_hbm, out_hbm, idx0, idx1, rows0, rows1, rt0, rt1,
          isem, gsem, wsem):
        idx_v = (idx0, idx1)
        rows_v = (rows0, rows1)
        rt_v = (rt0, rt1)
        wid = lax.axis_index("s") * NC + lax.axis_index("c")
        t0 = wid * t_per_w

        def task_jc(g):
            t = t0 + g
            return t // nblk, (t % nblk) * IB

        def fire_idx(g, b):
            j, i0 = task_jc(g)
            pltpu.async_copy(xt_hbm.at[j, pl.ds(i0, IB)], idx_v[b], isem)

        def fire_gather(b):
            pltpu.async_copy(tp_hbm.at[idx_v[b]], rows_v[b], gsem)

        def drain_rows(b):
            pltpu.make_async_copy(tp_hbm.at[pl.ds(0, IB)], rows_v[b],
                                  gsem).wait()

        def drain_wr(b):
            pltpu.make_async_copy(
                out_hbm.at[pl.ds(0, D_MODEL), pl.ds(0, IB)], rt_v[b],
                wsem).wait()

        def drain_idx(b):
            pltpu.make_async_copy(xt_hbm.at[0, pl.ds(0, IB)], idx_v[b],
                                  isem).wait()

        fire_idx(0, 0)
        fire_idx(1, 1)
        drain_idx(0)
        fire_gather(0)

        def outer(g2, carry):
            for b in range(NBUF):
                g = g2 * NBUF + b
                nb = (b + 1) % NBUF
                drain_rows(b)

                @pl.when(g + 2 < t_per_w)
                def _():
                    fire_idx(g + 2, b)

                @pl.when(g + 1 < t_per_w)
                def _():
                    drain_idx(nb)
                    fire_gather(nb)

                @pl.when(g >= NBUF)
                def _():
                    drain_wr(b)

                rows_b = rows_v[b]
                rt_b = rt_v[b]
                cvecs = [lax.iota(jnp.int32, LANES) + (cb * LANES)
                         for cb in range(D_MODEL // LANES)]

                @plsc.parallel_loop(0, IB, unroll=8)
                def tr_body(i):
                    ivec = jnp.full((LANES,), i, jnp.int32)
                    for cb in range(D_MODEL // LANES):
                        v = rows_b[i, pl.ds(cb * LANES, LANES)]
                        plsc.store_scatter(rt_b, [cvecs[cb], ivec],
                                           v * SCALE)

                j, i0 = task_jc(g)
                pltpu.async_copy(
                    rt_b,
                    out_hbm.at[pl.ds(j * D_MODEL, D_MODEL), pl.ds(i0, IB)],
                    wsem,
                )
            return carry

        lax.fori_loop(0, t_per_w // NBUF, outer, 0)
        drain_wr(0)
        drain_wr(1)

    return k


def kernel(x, table):
    N, S = x.shape
    xt = jnp.transpose(x).astype(jnp.int32)
    tp = jnp.pad(table, ((0, 0), (0, DPAD - D_MODEL)))
    out2 = _build(S, N)(xt, tp)
    out3 = out2.reshape(S, D_MODEL, N)
    return jnp.transpose(out3, (2, 0, 1))
